# baseline (speedup 1.0000x reference)
import functools

import jax
import jax.numpy as jnp
from jax import lax
from jax.experimental import pallas as pl
from jax.experimental.pallas import tpu as pltpu
from jax.experimental.pallas import tpu_sc as plsc

VOCAB = 1000000
DIM = 64
B = 16384
WIN = 20
NNEG = 20

NC = 2
NS = 16
LANES = 16
NW = NC * NS
CPW = B // NW
S = 16
NCHUNK = CPW // S
S20 = S * WIN
GI = 64
NGATH = S20 // GI
SROW = 2 * DIM
NJ = DIM // LANES

_mesh = plsc.VectorSubcoreMesh(core_axis_name="c", subcore_axis_name="s")


def _tree_sum(gs):
    while len(gs) > 1:
        nxt = [gs[i] + gs[i + 1] for i in range(0, len(gs) - 1, 2)]
        if len(gs) % 2:
            nxt.append(gs[-1])
        gs = nxt
    return gs[0]


@functools.partial(
    pl.kernel,
    out_type=[
        jax.ShapeDtypeStruct((B,), jnp.float32),
        jax.ShapeDtypeStruct((B * NNEG,), jnp.float32),
    ],
    name="cbow_scores",
    mesh=_mesh,
    compiler_params=pltpu.CompilerParams(
        needs_layout_passes=False, use_tc_tiling_on_sc=True),
    scratch_types=[
        pltpu.VMEM((CPW,), jnp.int32),
        pltpu.VMEM((WIN, CPW), jnp.int32),
        pltpu.VMEM((NNEG, CPW), jnp.int32),
        pltpu.VMEM((S,), jnp.int32),
        pltpu.VMEM((NGATH, GI), jnp.int32),
        pltpu.VMEM((NGATH, GI), jnp.int32),
        pltpu.VMEM((S20, SROW), jnp.float32),
        pltpu.VMEM((S20, SROW), jnp.float32),
        pltpu.VMEM((S, SROW), jnp.float32),
        pltpu.VMEM((CPW,), jnp.float32),
        pltpu.VMEM((CPW * NNEG,), jnp.float32),
        pltpu.SemaphoreType.DMA,
    ],
)
def _sc_scores(ctx_hbm, tgt_hbm, neg_hbm, wt_hbm, wc_hbm,
               pos_hbm, nout_hbm,
               cidx_v, tidx_v, nidx_v, cg_v, tg_v, ng_v,
               trows_v, nrows_v, crows_v, pos_v, nsc_v, sem):
    wid = lax.axis_index("s") * NC + lax.axis_index("c")

    pltpu.sync_copy(ctx_hbm.at[pl.ds(wid * CPW, CPW)], cidx_v)
    pltpu.sync_copy(tgt_hbm.at[pl.ds(0, WIN), pl.ds(wid * CPW, CPW)], tidx_v)
    pltpu.sync_copy(neg_hbm.at[pl.ds(0, NNEG), pl.ds(wid * CPW, CPW)], nidx_v)

    lane_iota = lax.iota(jnp.int32, LANES)

    def chunk_body(c, carry):
        for w in range(WIN):
            p = w * S
            tg_v[p // GI, pl.ds(p % GI, LANES)] = lax.shift_right_logical(
                tidx_v[w, pl.ds(c * S, LANES)], 1)
            ng_v[p // GI, pl.ds(p % GI, LANES)] = lax.shift_right_logical(
                nidx_v[w, pl.ds(c * S, LANES)], 1)
        cg_v[...] = lax.shift_right_logical(cidx_v[pl.ds(c * S, S)], 1)

        copies = []
        for j in range(NGATH):
            copies.append(pltpu.async_copy(
                wt_hbm.at[tg_v.at[j]], trows_v.at[pl.ds(j * GI, GI)], sem))
            copies.append(pltpu.async_copy(
                wc_hbm.at[ng_v.at[j]], nrows_v.at[pl.ds(j * GI, GI)], sem))
        copies.append(pltpu.async_copy(wc_hbm.at[cg_v], crows_v, sem))
        for cp in copies:
            cp.wait()

        zero = jnp.zeros((LANES,), jnp.float32)

        def ebody(il, scores):
            m = lane_iota == il
            zi = jnp.zeros((LANES,), jnp.int32)

            def par(vec):
                return (jnp.sum(jnp.where(m, vec, zi)) & 1) * DIM

            toff = [par(tidx_v[w, pl.ds(c * S, LANES)]) for w in range(WIN)]
            t = [trows_v[il, pl.ds(toff[0] + LANES * j, LANES)]
                 for j in range(NJ)]
            for w in range(1, WIN):
                rw = [trows_v[w * S + il, pl.ds(toff[w] + LANES * j, LANES)]
                      for j in range(NJ)]
                t = [t[j] + rw[j] for j in range(NJ)]
            coff = par(cidx_v[pl.ds(c * S, LANES)])
            pv = _tree_sum([t[j] * crows_v[il, pl.ds(coff + LANES * j, LANES)]
                            for j in range(NJ)])
            out = [jnp.where(m, jnp.full((LANES,), jnp.sum(pv)), scores[0])]
            for n in range(NNEG):
                noff = par(nidx_v[n, pl.ds(c * S, LANES)])
                nv = _tree_sum([
                    t[j] * nrows_v[n * S + il, pl.ds(noff + LANES * j, LANES)]
                    for j in range(NJ)])
                out.append(jnp.where(
                    m, jnp.full((LANES,), jnp.sum(nv)), scores[1 + n]))
            return tuple(out)

        res = lax.fori_loop(0, LANES, ebody, (zero,) * (1 + NNEG))
        pos_v[pl.ds(c * S, S)] = res[0]
        for n in range(NNEG):
            nsc_v[pl.ds(c * S20 + n * S, S)] = res[1 + n]
        return carry

    lax.fori_loop(0, NCHUNK, chunk_body, 0)

    pltpu.sync_copy(pos_v, pos_hbm.at[pl.ds(wid * CPW, CPW)])
    pltpu.sync_copy(nsc_v, nout_hbm.at[pl.ds(wid * CPW * NNEG, CPW * NNEG)])


def _loss_body(pos_ref, neg_ref, out_ref):
    p = jnp.clip(pos_ref[...] * (1.0 / WIN), -10.0, 10.0)
    n = jnp.clip(neg_ref[...] * (1.0 / WIN), -10.0, 10.0)
    lp = jnp.sum(jnp.log1p(jnp.exp(-p)))
    ln = jnp.sum(jnp.log1p(jnp.exp(n)))
    out_ref[...] = ((lp + ln) * (1.0 / B)).reshape(1, 1)


_loss_tc = pl.pallas_call(
    _loss_body,
    out_shape=jax.ShapeDtypeStruct((1, 1), jnp.float32),
)


@jax.jit
def kernel(context, target, negatives, W_target, W_context):
    wt2 = W_target.reshape(VOCAB // 2, 2 * DIM)
    wc2 = W_context.reshape(VOCAB // 2, 2 * DIM)
    pos_raw, neg_raw = _sc_scores(context, target.T, negatives.T, wt2, wc2)
    out = _loss_tc(pos_raw.reshape(128, 128), neg_raw.reshape(-1, 128))
    return out[0, 0]
